# Initial kernel scaffold; baseline (speedup 1.0000x reference)
import jax
import jax.numpy as jnp
from jax.experimental import pallas as pl


def kernel(x, edge_index, batch, params):
    raise NotImplementedError("write your pallas kernel here")


# speedup vs baseline: 3.8565x; 3.8565x over previous
import functools

import jax
import jax.numpy as jnp
from jax import lax
from jax.experimental import pallas as pl
from jax.experimental.pallas import tpu as pltpu
from jax.experimental.pallas import tpu_sc as plsc

N = 10000
E = 160000
F = 256
H = 64
C = 10
G = 128

NC = 2
NS = 16
NW = NC * NS
CH = 128
EPW = 5120
NCHUNK = EPW // CH
EPAD = NW * EPW
ACC_ROWS = N + 16
RS = 624
RSREM = N - NS * RS

PW = 16
ONES_COL = 10
NPOOL_PAD = 10240
POOL_CH = 64
POOL_RPW = NPOOL_PAD // NW
POOL_NCHUNK = POOL_RPW // POOL_CH
POOL_ACC_ROWS = G + 16


@functools.lru_cache(maxsize=None)
def _sc_kernels():
    mesh = plsc.VectorSubcoreMesh(core_axis_name="c", subcore_axis_name="s")
    cp = pltpu.CompilerParams(use_tc_tiling_on_sc=False)

    @functools.partial(
        pl.kernel,
        out_type=jax.ShapeDtypeStruct((NC, N, H), jnp.float32),
        mesh=mesh,
        compiler_params=cp,
        scratch_types=[
            pltpu.VMEM((1, CH), jnp.int32),
            pltpu.VMEM((1, CH), jnp.int32),
            pltpu.VMEM((CH, H), jnp.float32),
            pltpu.VMEM_SHARED((ACC_ROWS, H), jnp.float32),
            pltpu.SemaphoreType.DMA,
        ],
    )
    def agg_sc(z_hbm, src_hbm, dst_hbm, out_hbm, sidx, didx, rows, acc, sem):
        cid = lax.axis_index("c")
        sid = lax.axis_index("s")
        wid = sid * NC + cid
        r0 = sid * RS
        pltpu.sync_copy(z_hbm.at[pl.ds(r0, RS)], acc.at[pl.ds(r0, RS)])

        @pl.when(sid == NS - 1)
        def _():
            pltpu.sync_copy(z_hbm.at[pl.ds(NS * RS, RSREM)],
                            acc.at[pl.ds(NS * RS, RSREM)])

        plsc.subcore_barrier()

        @pl.loop(0, NCHUNK)
        def _(j):
            pltpu.sync_copy(src_hbm.at[wid, j], sidx)
            pltpu.sync_copy(dst_hbm.at[wid, j], didx)
            pltpu.async_copy(z_hbm.at[sidx.at[0]], rows, sem).wait()
            pltpu.sync_copy(rows, acc.at[didx.at[0]], add=True)

        plsc.subcore_barrier()
        pltpu.sync_copy(acc.at[pl.ds(r0, RS)],
                        out_hbm.at[cid, pl.ds(r0, RS)])

        @pl.when(sid == NS - 1)
        def _():
            pltpu.sync_copy(acc.at[pl.ds(NS * RS, RSREM)],
                            out_hbm.at[cid, pl.ds(NS * RS, RSREM)])

    @functools.partial(
        pl.kernel,
        out_type=jax.ShapeDtypeStruct((NC, POOL_ACC_ROWS, PW), jnp.float32),
        mesh=mesh,
        compiler_params=cp,
        scratch_types=[
            pltpu.VMEM((1, POOL_CH), jnp.int32),
            pltpu.VMEM((POOL_CH, PW), jnp.float32),
            pltpu.VMEM((16, PW), jnp.float32),
            pltpu.VMEM_SHARED((POOL_ACC_ROWS, PW), jnp.float32),
            pltpu.SemaphoreType.DMA,
        ],
    )
    def pool_sc(xf_hbm, b_hbm, out_hbm, bidx, rows, zbuf, acc, sem):
        cid = lax.axis_index("c")
        sid = lax.axis_index("s")
        wid = sid * NC + cid
        a0 = sid * 8

        @pl.loop(0, 16)
        def _(i):
            zbuf[i] = jnp.zeros((PW,), jnp.float32)

        pltpu.sync_copy(zbuf.at[pl.ds(0, 8)], acc.at[pl.ds(a0, 8)])

        @pl.when(sid == 0)
        def _():
            pltpu.sync_copy(zbuf, acc.at[pl.ds(G, 16)])

        plsc.subcore_barrier()

        @pl.loop(0, POOL_NCHUNK)
        def _(j):
            base = wid * POOL_RPW + j * POOL_CH
            pltpu.sync_copy(xf_hbm.at[pl.ds(base, POOL_CH)], rows)
            pltpu.sync_copy(b_hbm.at[wid, j], bidx)
            pltpu.sync_copy(rows, acc.at[bidx.at[0]], add=True)

        plsc.subcore_barrier()
        pltpu.sync_copy(acc.at[pl.ds(a0, 8)], out_hbm.at[cid, pl.ds(a0, 8)])

        @pl.when(sid == 0)
        def _():
            pltpu.sync_copy(acc.at[pl.ds(G, 16)],
                            out_hbm.at[cid, pl.ds(G, 16)])

    return agg_sc, pool_sc


_PREC = lax.Precision.HIGHEST


def _dot(a, b):
    return jnp.dot(a, b, precision=_PREC, preferred_element_type=jnp.float32)


def _bn_relu(h, g, b):
    m = jnp.mean(h, axis=0, keepdims=True)
    c = h - m
    v = jnp.mean(c * c, axis=0, keepdims=True)
    return jnp.maximum(c / jnp.sqrt(v + 1e-5) * g + b, 0.0)


def _stage_a_body(x_ref, wa_ref, wb_ref, z1_ref, xp_ref):
    x = x_ref[...]
    z1_ref[...] = _dot(x, wa_ref[...])
    xp_ref[...] = _dot(x, wb_ref[...])


def _p1_body(z_ref, pp_ref, b1, g1, be1, w2, b2, t_ref):
    s = pp_ref[0] + pp_ref[1] - z_ref[...] + b1[...]
    a = _bn_relu(s, g1[...], be1[...])
    t_ref[...] = _dot(a, w2[...]) + b2[...]


def _p2_b_body(t_ref, g2, be2, wn, z2_ref):
    h = _bn_relu(t_ref[...], g2[...], be2[...])
    z2_ref[...] = _dot(h, wn[...])


def _p2_c_body(t_ref, g2, be2, xp_ref, wb, wnb, z3_ref, x2w_ref):
    h = _bn_relu(t_ref[...], g2[...], be2[...])
    z3_ref[...] = xp_ref[...] + _dot(h, wb[...])
    x2w_ref[...] = _dot(h, wnb[...])


def _p2_d_body(t_ref, g2, be2, x2w_ref, wa, z4_ref):
    h = _bn_relu(t_ref[...], g2[...], be2[...])
    z4_ref[...] = _dot(h, wa[...]) + x2w_ref[...]


def _p2_e_body(t_ref, g2, be2, wf, bf, xf_ref):
    h = _bn_relu(t_ref[...], g2[...], be2[...])
    xf_ref[0:N, :] = _dot(h, wf[...]) + bf[...]
    xf_ref[N:NPOOL_PAD, :] = jnp.zeros((NPOOL_PAD - N, PW), jnp.float32)


def _stage_f_body(pp_ref, out_ref):
    s = pp_ref[0] + pp_ref[1]
    cnt = jnp.maximum(s[:G, ONES_COL:ONES_COL + 1], 1.0)
    out_ref[...] = s[:G, :C] / cnt


def _f32(shape):
    return jax.ShapeDtypeStruct(shape, jnp.float32)


def kernel(x, edge_index, batch, params):
    src = edge_index[0].astype(jnp.int32)
    dst = edge_index[1].astype(jnp.int32)
    src3 = jnp.concatenate(
        [src, jnp.zeros((EPAD - E,), jnp.int32)]).reshape(NW, NCHUNK, 1, CH)
    dst3 = jnp.concatenate(
        [dst, jnp.full((EPAD - E,), N, jnp.int32)]).reshape(NW, NCHUNK, 1, CH)
    batch3 = jnp.concatenate(
        [batch.astype(jnp.int32), jnp.full((NPOOL_PAD - N,), G, jnp.int32)]
    ).reshape(NW, POOL_NCHUNK, 1, POOL_CH)

    dc1, dc2, uc1, uc2 = (params['dc1'], params['dc2'], params['uc1'],
                          params['uc2'])

    def v1(p):
        return (p['b1'].reshape(1, H), p['g1'].reshape(1, H),
                p['be1'].reshape(1, H), p['W2'], p['b2'].reshape(1, H))

    def v2(p):
        return (p['g2'].reshape(1, H), p['be2'].reshape(1, H))

    agg_sc, pool_sc = _sc_kernels()

    wf = jnp.zeros((H, PW), jnp.float32).at[:, :C].set(params['final']['W'])
    bf = jnp.zeros((1, PW), jnp.float32).at[0, :C].set(
        params['final']['b']).at[0, ONES_COL].set(1.0)

    z1, xp = pl.pallas_call(
        _stage_a_body, out_shape=[_f32((N, H)), _f32((N, H))],
    )(x, dc1['W1'], uc1['W1'][:F])

    pp1 = agg_sc(z1, src3, dst3)
    t1 = pl.pallas_call(
        _p1_body, out_shape=_f32((N, H)))(z1, pp1, *v1(dc1))
    z2 = pl.pallas_call(
        _p2_b_body, out_shape=_f32((N, H)))(t1, *v2(dc1), dc2['W1'])

    pp2 = agg_sc(z2, src3, dst3)
    t2 = pl.pallas_call(
        _p1_body, out_shape=_f32((N, H)))(z2, pp2, *v1(dc2))
    z3, x2w = pl.pallas_call(
        _p2_c_body, out_shape=[_f32((N, H)), _f32((N, H))],
    )(t2, *v2(dc2), xp, uc1['W1'][F:], uc2['W1'][H:])

    pp3 = agg_sc(z3, src3, dst3)
    t3 = pl.pallas_call(
        _p1_body, out_shape=_f32((N, H)))(z3, pp3, *v1(uc1))
    z4 = pl.pallas_call(
        _p2_d_body, out_shape=_f32((N, H)))(t3, *v2(uc1), x2w, uc2['W1'][:H])

    pp4 = agg_sc(z4, src3, dst3)
    t4 = pl.pallas_call(
        _p1_body, out_shape=_f32((N, H)))(z4, pp4, *v1(uc2))
    xf = pl.pallas_call(
        _p2_e_body, out_shape=_f32((NPOOL_PAD, PW)))(t4, *v2(uc2), wf, bf)

    ppool = pool_sc(xf, batch3)
    return pl.pallas_call(_stage_f_body, out_shape=_f32((G, C)))(ppool)


# speedup vs baseline: 4.6937x; 1.2171x over previous
import functools

import jax
import jax.numpy as jnp
from jax import lax
from jax.experimental import pallas as pl
from jax.experimental.pallas import tpu as pltpu
from jax.experimental.pallas import tpu_sc as plsc

N = 10000
E = 160000
F = 256
H = 64
C = 10
G = 128

NC = 2
NS = 16
NW = NC * NS
CH = 128
EPW = 5120
NCHUNK = EPW // CH
EPAD = NW * EPW
NBUF = 4
ACC_ROWS = N + 16
RS = 624
RSREM = N - NS * RS

PW = 16
ONES_COL = 10
NPOOL_PAD = 10240
POOL_CH = 64
POOL_RPW = NPOOL_PAD // NW
POOL_NCHUNK = POOL_RPW // POOL_CH
POOL_ACC_ROWS = G + 16


@functools.lru_cache(maxsize=None)
def _sc_kernels():
    mesh = plsc.VectorSubcoreMesh(core_axis_name="c", subcore_axis_name="s")
    cp = pltpu.CompilerParams(use_tc_tiling_on_sc=False)

    @functools.partial(
        pl.kernel,
        out_type=jax.ShapeDtypeStruct((NC, N, H), jnp.float32),
        mesh=mesh,
        compiler_params=cp,
        scratch_types=[
            pltpu.VMEM((NBUF, 1, CH), jnp.int32),
            pltpu.VMEM((NBUF, 1, CH), jnp.int32),
            pltpu.VMEM((NBUF, CH, H), jnp.float32),
            pltpu.VMEM_SHARED((ACC_ROWS, H), jnp.float32),
            pltpu.SemaphoreType.DMA((NBUF,)),
            pltpu.SemaphoreType.DMA((NBUF,)),
            pltpu.SemaphoreType.DMA((NBUF,)),
            pltpu.SemaphoreType.DMA((NBUF,)),
        ],
    )
    def agg_sc(z_hbm, src_hbm, dst_hbm, out_hbm, sidx, didx, rows, acc,
               isem, dsem, gsem, ssem):
        cid = lax.axis_index("c")
        sid = lax.axis_index("s")
        wid = sid * NC + cid
        r0 = sid * RS
        pltpu.sync_copy(z_hbm.at[pl.ds(r0, RS)], acc.at[pl.ds(r0, RS)])

        @pl.when(sid == NS - 1)
        def _():
            pltpu.sync_copy(z_hbm.at[pl.ds(NS * RS, RSREM)],
                            acc.at[pl.ds(NS * RS, RSREM)])

        plsc.subcore_barrier()

        for b in range(NBUF):
            pltpu.async_copy(src_hbm.at[wid, b], sidx.at[b], isem.at[b])

        @pl.loop(0, NCHUNK, step=NBUF)
        def _(j0):
            for b in range(NBUF):
                @pl.when(j0 > 0)
                def _():
                    pltpu.make_async_copy(
                        z_hbm.at[pl.ds(0, CH)], rows.at[b], ssem.at[b]).wait()

                pltpu.async_copy(dst_hbm.at[wid, j0 + b], didx.at[b],
                                 dsem.at[b])
                pltpu.make_async_copy(
                    src_hbm.at[wid, 0], sidx.at[b], isem.at[b]).wait()
                pltpu.async_copy(z_hbm.at[sidx.at[b, 0]], rows.at[b],
                                 gsem.at[b])
                pltpu.make_async_copy(
                    z_hbm.at[pl.ds(0, CH)], rows.at[b], gsem.at[b]).wait()

                @pl.when(j0 < NCHUNK - NBUF)
                def _():
                    pltpu.async_copy(src_hbm.at[wid, j0 + b + NBUF],
                                     sidx.at[b], isem.at[b])

                pltpu.make_async_copy(
                    dst_hbm.at[wid, 0], didx.at[b], dsem.at[b]).wait()
                pltpu.async_copy(rows.at[b], acc.at[didx.at[b, 0]],
                                 ssem.at[b], add=True)

        for b in range(NBUF):
            pltpu.make_async_copy(
                z_hbm.at[pl.ds(0, CH)], rows.at[b], ssem.at[b]).wait()

        plsc.subcore_barrier()
        pltpu.sync_copy(acc.at[pl.ds(r0, RS)],
                        out_hbm.at[cid, pl.ds(r0, RS)])

        @pl.when(sid == NS - 1)
        def _():
            pltpu.sync_copy(acc.at[pl.ds(NS * RS, RSREM)],
                            out_hbm.at[cid, pl.ds(NS * RS, RSREM)])

    @functools.partial(
        pl.kernel,
        out_type=jax.ShapeDtypeStruct((NC, POOL_ACC_ROWS, PW), jnp.float32),
        mesh=mesh,
        compiler_params=cp,
        scratch_types=[
            pltpu.VMEM((1, POOL_CH), jnp.int32),
            pltpu.VMEM((POOL_CH, PW), jnp.float32),
            pltpu.VMEM((16, PW), jnp.float32),
            pltpu.VMEM_SHARED((POOL_ACC_ROWS, PW), jnp.float32),
            pltpu.SemaphoreType.DMA,
        ],
    )
    def pool_sc(xf_hbm, b_hbm, out_hbm, bidx, rows, zbuf, acc, sem):
        cid = lax.axis_index("c")
        sid = lax.axis_index("s")
        wid = sid * NC + cid
        a0 = sid * 8

        @pl.loop(0, 16)
        def _(i):
            zbuf[i] = jnp.zeros((PW,), jnp.float32)

        pltpu.sync_copy(zbuf.at[pl.ds(0, 8)], acc.at[pl.ds(a0, 8)])

        @pl.when(sid == 0)
        def _():
            pltpu.sync_copy(zbuf, acc.at[pl.ds(G, 16)])

        plsc.subcore_barrier()

        @pl.loop(0, POOL_NCHUNK)
        def _(j):
            base = wid * POOL_RPW + j * POOL_CH
            pltpu.sync_copy(xf_hbm.at[pl.ds(base, POOL_CH)], rows)
            pltpu.sync_copy(b_hbm.at[wid, j], bidx)
            pltpu.sync_copy(rows, acc.at[bidx.at[0]], add=True)

        plsc.subcore_barrier()
        pltpu.sync_copy(acc.at[pl.ds(a0, 8)], out_hbm.at[cid, pl.ds(a0, 8)])

        @pl.when(sid == 0)
        def _():
            pltpu.sync_copy(acc.at[pl.ds(G, 16)],
                            out_hbm.at[cid, pl.ds(G, 16)])

    return agg_sc, pool_sc


_PREC = lax.Precision.HIGHEST


def _dot(a, b):
    return jnp.dot(a, b, precision=_PREC, preferred_element_type=jnp.float32)


def _bn_relu(h, g, b):
    m = jnp.mean(h, axis=0, keepdims=True)
    c = h - m
    v = jnp.mean(c * c, axis=0, keepdims=True)
    return jnp.maximum(c / jnp.sqrt(v + 1e-5) * g + b, 0.0)


def _stage_a_body(x_ref, wa_ref, wb_ref, z1_ref, xp_ref):
    x = x_ref[...]
    z1_ref[...] = _dot(x, wa_ref[...])
    xp_ref[...] = _dot(x, wb_ref[...])


def _p1_body(z_ref, pp_ref, b1, g1, be1, w2, b2, t_ref):
    s = pp_ref[0] + pp_ref[1] - z_ref[...] + b1[...]
    a = _bn_relu(s, g1[...], be1[...])
    t_ref[...] = _dot(a, w2[...]) + b2[...]


def _p2_b_body(t_ref, g2, be2, wn, z2_ref):
    h = _bn_relu(t_ref[...], g2[...], be2[...])
    z2_ref[...] = _dot(h, wn[...])


def _p2_c_body(t_ref, g2, be2, xp_ref, wb, wnb, z3_ref, x2w_ref):
    h = _bn_relu(t_ref[...], g2[...], be2[...])
    z3_ref[...] = xp_ref[...] + _dot(h, wb[...])
    x2w_ref[...] = _dot(h, wnb[...])


def _p2_d_body(t_ref, g2, be2, x2w_ref, wa, z4_ref):
    h = _bn_relu(t_ref[...], g2[...], be2[...])
    z4_ref[...] = _dot(h, wa[...]) + x2w_ref[...]


def _p2_e_body(t_ref, g2, be2, wf, bf, xf_ref):
    h = _bn_relu(t_ref[...], g2[...], be2[...])
    xf_ref[0:N, :] = _dot(h, wf[...]) + bf[...]
    xf_ref[N:NPOOL_PAD, :] = jnp.zeros((NPOOL_PAD - N, PW), jnp.float32)


def _stage_f_body(pp_ref, out_ref):
    s = pp_ref[0] + pp_ref[1]
    cnt = jnp.maximum(s[:G, ONES_COL:ONES_COL + 1], 1.0)
    out_ref[...] = s[:G, :C] / cnt


def _f32(shape):
    return jax.ShapeDtypeStruct(shape, jnp.float32)


def kernel(x, edge_index, batch, params):
    src = edge_index[0].astype(jnp.int32)
    dst = edge_index[1].astype(jnp.int32)
    src3 = jnp.concatenate(
        [src, jnp.zeros((EPAD - E,), jnp.int32)]).reshape(NW, NCHUNK, 1, CH)
    dst3 = jnp.concatenate(
        [dst, jnp.full((EPAD - E,), N, jnp.int32)]).reshape(NW, NCHUNK, 1, CH)
    batch3 = jnp.concatenate(
        [batch.astype(jnp.int32), jnp.full((NPOOL_PAD - N,), G, jnp.int32)]
    ).reshape(NW, POOL_NCHUNK, 1, POOL_CH)

    dc1, dc2, uc1, uc2 = (params['dc1'], params['dc2'], params['uc1'],
                          params['uc2'])

    def v1(p):
        return (p['b1'].reshape(1, H), p['g1'].reshape(1, H),
                p['be1'].reshape(1, H), p['W2'], p['b2'].reshape(1, H))

    def v2(p):
        return (p['g2'].reshape(1, H), p['be2'].reshape(1, H))

    agg_sc, pool_sc = _sc_kernels()

    wf = jnp.zeros((H, PW), jnp.float32).at[:, :C].set(params['final']['W'])
    bf = jnp.zeros((1, PW), jnp.float32).at[0, :C].set(
        params['final']['b']).at[0, ONES_COL].set(1.0)

    z1, xp = pl.pallas_call(
        _stage_a_body, out_shape=[_f32((N, H)), _f32((N, H))],
    )(x, dc1['W1'], uc1['W1'][:F])

    pp1 = agg_sc(z1, src3, dst3)
    t1 = pl.pallas_call(
        _p1_body, out_shape=_f32((N, H)))(z1, pp1, *v1(dc1))
    z2 = pl.pallas_call(
        _p2_b_body, out_shape=_f32((N, H)))(t1, *v2(dc1), dc2['W1'])

    pp2 = agg_sc(z2, src3, dst3)
    t2 = pl.pallas_call(
        _p1_body, out_shape=_f32((N, H)))(z2, pp2, *v1(dc2))
    z3, x2w = pl.pallas_call(
        _p2_c_body, out_shape=[_f32((N, H)), _f32((N, H))],
    )(t2, *v2(dc2), xp, uc1['W1'][F:], uc2['W1'][H:])

    pp3 = agg_sc(z3, src3, dst3)
    t3 = pl.pallas_call(
        _p1_body, out_shape=_f32((N, H)))(z3, pp3, *v1(uc1))
    z4 = pl.pallas_call(
        _p2_d_body, out_shape=_f32((N, H)))(t3, *v2(uc1), x2w, uc2['W1'][:H])

    pp4 = agg_sc(z4, src3, dst3)
    t4 = pl.pallas_call(
        _p1_body, out_shape=_f32((N, H)))(z4, pp4, *v1(uc2))
    xf = pl.pallas_call(
        _p2_e_body, out_shape=_f32((NPOOL_PAD, PW)))(t4, *v2(uc2), wf, bf)

    ppool = pool_sc(xf, batch3)
    return pl.pallas_call(_stage_f_body, out_shape=_f32((G, C)))(ppool)


# speedup vs baseline: 4.9718x; 1.0592x over previous
import functools

import jax
import jax.numpy as jnp
from jax import lax
from jax.experimental import pallas as pl
from jax.experimental.pallas import tpu as pltpu
from jax.experimental.pallas import tpu_sc as plsc

N = 10000
E = 160000
F = 256
H = 64
C = 10
G = 128

NC = 2
NS = 16
NW = NC * NS
CH = 128
EPW = 5120
NCHUNK = EPW // CH
EPAD = NW * EPW
NBUF = 4
NRING = 2 * NBUF
ACC_ROWS = N + 16
RS = 624
RSREM = N - NS * RS

PW = 16
ONES_COL = 10
NPOOL_PAD = 10240
POOL_CH = 64
POOL_RPW = NPOOL_PAD // NW
POOL_NCHUNK = POOL_RPW // POOL_CH
POOL_ACC_ROWS = G + 16


@functools.lru_cache(maxsize=None)
def _sc_kernels():
    mesh = plsc.VectorSubcoreMesh(core_axis_name="c", subcore_axis_name="s")
    cp = pltpu.CompilerParams(use_tc_tiling_on_sc=False)

    @functools.partial(
        pl.kernel,
        out_type=jax.ShapeDtypeStruct((NC, N, H), jnp.float32),
        mesh=mesh,
        compiler_params=cp,
        scratch_types=[
            pltpu.VMEM((NCHUNK, 1, CH), jnp.int32),
            pltpu.VMEM((NCHUNK, 1, CH), jnp.int32),
            pltpu.VMEM((NRING, CH, H), jnp.float32),
            pltpu.VMEM_SHARED((ACC_ROWS, H), jnp.float32),
            pltpu.SemaphoreType.DMA((2,)),
            pltpu.SemaphoreType.DMA((NRING,)),
            pltpu.SemaphoreType.DMA((NRING,)),
        ],
    )
    def agg_sc(z_hbm, src_hbm, dst_hbm, out_hbm, sidx, didx, rows, acc,
               isem, gsem, ssem):
        cid = lax.axis_index("c")
        sid = lax.axis_index("s")
        wid = sid * NC + cid
        r0 = sid * RS
        pltpu.async_copy(src_hbm.at[wid], sidx, isem.at[0])
        pltpu.async_copy(dst_hbm.at[wid], didx, isem.at[1])
        pltpu.sync_copy(z_hbm.at[pl.ds(r0, RS)], acc.at[pl.ds(r0, RS)])

        @pl.when(sid == NS - 1)
        def _():
            pltpu.sync_copy(z_hbm.at[pl.ds(NS * RS, RSREM)],
                            acc.at[pl.ds(NS * RS, RSREM)])

        plsc.subcore_barrier()
        pltpu.make_async_copy(src_hbm.at[wid], sidx, isem.at[0]).wait()
        pltpu.make_async_copy(dst_hbm.at[wid], didx, isem.at[1]).wait()

        for b in range(NBUF):
            pltpu.async_copy(z_hbm.at[sidx.at[b, 0]], rows.at[b], gsem.at[b])

        @pl.loop(0, NCHUNK, step=NRING)
        def _(j0):
            for b in range(NRING):
                j = j0 + b
                pltpu.make_async_copy(
                    z_hbm.at[pl.ds(0, CH)], rows.at[b], gsem.at[b]).wait()
                pltpu.async_copy(rows.at[b], acc.at[didx.at[j, 0]],
                                 ssem.at[b], add=True)
                b2 = (b + NBUF) % NRING
                if b2 < NBUF:
                    pltpu.make_async_copy(
                        z_hbm.at[pl.ds(0, CH)], rows.at[b2],
                        ssem.at[b2]).wait()
                else:
                    @pl.when(j0 > 0)
                    def _():
                        pltpu.make_async_copy(
                            z_hbm.at[pl.ds(0, CH)], rows.at[b2],
                            ssem.at[b2]).wait()

                @pl.when(j + NBUF < NCHUNK)
                def _():
                    pltpu.async_copy(z_hbm.at[sidx.at[j + NBUF, 0]],
                                     rows.at[b2], gsem.at[b2])

        for b in range(NBUF, NRING):
            pltpu.make_async_copy(
                z_hbm.at[pl.ds(0, CH)], rows.at[b], ssem.at[b]).wait()

        plsc.subcore_barrier()
        pltpu.sync_copy(acc.at[pl.ds(r0, RS)],
                        out_hbm.at[cid, pl.ds(r0, RS)])

        @pl.when(sid == NS - 1)
        def _():
            pltpu.sync_copy(acc.at[pl.ds(NS * RS, RSREM)],
                            out_hbm.at[cid, pl.ds(NS * RS, RSREM)])

    @functools.partial(
        pl.kernel,
        out_type=jax.ShapeDtypeStruct((NC, POOL_ACC_ROWS, PW), jnp.float32),
        mesh=mesh,
        compiler_params=cp,
        scratch_types=[
            pltpu.VMEM((1, POOL_CH), jnp.int32),
            pltpu.VMEM((POOL_CH, PW), jnp.float32),
            pltpu.VMEM((16, PW), jnp.float32),
            pltpu.VMEM_SHARED((POOL_ACC_ROWS, PW), jnp.float32),
            pltpu.SemaphoreType.DMA,
        ],
    )
    def pool_sc(xf_hbm, b_hbm, out_hbm, bidx, rows, zbuf, acc, sem):
        cid = lax.axis_index("c")
        sid = lax.axis_index("s")
        wid = sid * NC + cid
        a0 = sid * 8

        @pl.loop(0, 16)
        def _(i):
            zbuf[i] = jnp.zeros((PW,), jnp.float32)

        pltpu.sync_copy(zbuf.at[pl.ds(0, 8)], acc.at[pl.ds(a0, 8)])

        @pl.when(sid == 0)
        def _():
            pltpu.sync_copy(zbuf, acc.at[pl.ds(G, 16)])

        plsc.subcore_barrier()

        @pl.loop(0, POOL_NCHUNK)
        def _(j):
            base = wid * POOL_RPW + j * POOL_CH
            pltpu.sync_copy(xf_hbm.at[pl.ds(base, POOL_CH)], rows)
            pltpu.sync_copy(b_hbm.at[wid, j], bidx)
            pltpu.sync_copy(rows, acc.at[bidx.at[0]], add=True)

        plsc.subcore_barrier()
        pltpu.sync_copy(acc.at[pl.ds(a0, 8)], out_hbm.at[cid, pl.ds(a0, 8)])

        @pl.when(sid == 0)
        def _():
            pltpu.sync_copy(acc.at[pl.ds(G, 16)],
                            out_hbm.at[cid, pl.ds(G, 16)])

    return agg_sc, pool_sc


_PREC = lax.Precision.HIGHEST


def _dot(a, b):
    return jnp.dot(a, b, precision=_PREC, preferred_element_type=jnp.float32)


def _bn_relu(h, g, b):
    m = jnp.mean(h, axis=0, keepdims=True)
    c = h - m
    v = jnp.mean(c * c, axis=0, keepdims=True)
    return jnp.maximum(c / jnp.sqrt(v + 1e-5) * g + b, 0.0)


def _stage_a_body(x_ref, wa_ref, wb_ref, z1_ref, xp_ref):
    x = x_ref[...]
    z1_ref[...] = _dot(x, wa_ref[...])
    xp_ref[...] = _dot(x, wb_ref[...])


def _p1_body(z_ref, pp_ref, b1, g1, be1, w2, b2, t_ref):
    s = pp_ref[0] + pp_ref[1] - z_ref[...] + b1[...]
    a = _bn_relu(s, g1[...], be1[...])
    t_ref[...] = _dot(a, w2[...]) + b2[...]


def _p2_b_body(t_ref, g2, be2, wn, z2_ref):
    h = _bn_relu(t_ref[...], g2[...], be2[...])
    z2_ref[...] = _dot(h, wn[...])


def _p2_c_body(t_ref, g2, be2, xp_ref, wb, wnb, z3_ref, x2w_ref):
    h = _bn_relu(t_ref[...], g2[...], be2[...])
    z3_ref[...] = xp_ref[...] + _dot(h, wb[...])
    x2w_ref[...] = _dot(h, wnb[...])


def _p2_d_body(t_ref, g2, be2, x2w_ref, wa, z4_ref):
    h = _bn_relu(t_ref[...], g2[...], be2[...])
    z4_ref[...] = _dot(h, wa[...]) + x2w_ref[...]


def _p2_e_body(t_ref, g2, be2, wf, bf, xf_ref):
    h = _bn_relu(t_ref[...], g2[...], be2[...])
    xf_ref[0:N, :] = _dot(h, wf[...]) + bf[...]
    xf_ref[N:NPOOL_PAD, :] = jnp.zeros((NPOOL_PAD - N, PW), jnp.float32)


def _stage_f_body(pp_ref, out_ref):
    s = pp_ref[0] + pp_ref[1]
    cnt = jnp.maximum(s[:G, ONES_COL:ONES_COL + 1], 1.0)
    out_ref[...] = s[:G, :C] / cnt


def _f32(shape):
    return jax.ShapeDtypeStruct(shape, jnp.float32)


def kernel(x, edge_index, batch, params):
    src = edge_index[0].astype(jnp.int32)
    dst = edge_index[1].astype(jnp.int32)
    src3 = jnp.concatenate(
        [src, jnp.zeros((EPAD - E,), jnp.int32)]).reshape(NW, NCHUNK, 1, CH)
    dst3 = jnp.concatenate(
        [dst, jnp.full((EPAD - E,), N, jnp.int32)]).reshape(NW, NCHUNK, 1, CH)
    batch3 = jnp.concatenate(
        [batch.astype(jnp.int32), jnp.full((NPOOL_PAD - N,), G, jnp.int32)]
    ).reshape(NW, POOL_NCHUNK, 1, POOL_CH)

    dc1, dc2, uc1, uc2 = (params['dc1'], params['dc2'], params['uc1'],
                          params['uc2'])

    def v1(p):
        return (p['b1'].reshape(1, H), p['g1'].reshape(1, H),
                p['be1'].reshape(1, H), p['W2'], p['b2'].reshape(1, H))

    def v2(p):
        return (p['g2'].reshape(1, H), p['be2'].reshape(1, H))

    agg_sc, pool_sc = _sc_kernels()

    wf = jnp.zeros((H, PW), jnp.float32).at[:, :C].set(params['final']['W'])
    bf = jnp.zeros((1, PW), jnp.float32).at[0, :C].set(
        params['final']['b']).at[0, ONES_COL].set(1.0)

    z1, xp = pl.pallas_call(
        _stage_a_body, out_shape=[_f32((N, H)), _f32((N, H))],
    )(x, dc1['W1'], uc1['W1'][:F])

    pp1 = agg_sc(z1, src3, dst3)
    t1 = pl.pallas_call(
        _p1_body, out_shape=_f32((N, H)))(z1, pp1, *v1(dc1))
    z2 = pl.pallas_call(
        _p2_b_body, out_shape=_f32((N, H)))(t1, *v2(dc1), dc2['W1'])

    pp2 = agg_sc(z2, src3, dst3)
    t2 = pl.pallas_call(
        _p1_body, out_shape=_f32((N, H)))(z2, pp2, *v1(dc2))
    z3, x2w = pl.pallas_call(
        _p2_c_body, out_shape=[_f32((N, H)), _f32((N, H))],
    )(t2, *v2(dc2), xp, uc1['W1'][F:], uc2['W1'][H:])

    pp3 = agg_sc(z3, src3, dst3)
    t3 = pl.pallas_call(
        _p1_body, out_shape=_f32((N, H)))(z3, pp3, *v1(uc1))
    z4 = pl.pallas_call(
        _p2_d_body, out_shape=_f32((N, H)))(t3, *v2(uc1), x2w, uc2['W1'][:H])

    pp4 = agg_sc(z4, src3, dst3)
    t4 = pl.pallas_call(
        _p1_body, out_shape=_f32((N, H)))(z4, pp4, *v1(uc2))
    xf = pl.pallas_call(
        _p2_e_body, out_shape=_f32((NPOOL_PAD, PW)))(t4, *v2(uc2), wf, bf)

    ppool = pool_sc(xf, batch3)
    return pl.pallas_call(_stage_f_body, out_shape=_f32((G, C)))(ppool)


# speedup vs baseline: 5.6827x; 1.1430x over previous
import functools

import jax
import jax.numpy as jnp
from jax import lax
from jax.experimental import pallas as pl
from jax.experimental.pallas import tpu as pltpu
from jax.experimental.pallas import tpu_sc as plsc

N = 10000
E = 160000
F = 256
H = 64
C = 10
G = 128

NC = 2
NS = 16
NW = NC * NS
CH = 128
EPW = 5120
NCHUNK = EPW // CH
EPAD = NW * EPW
NBUF = 4
NRING = 2 * NBUF
ACC_ROWS = N + 16
RS = 624
RSREM = N - NS * RS

PW = 16
ONES_COL = 10
NPOOL_PAD = 10240
POOL_CH = 64
POOL_RPW = NPOOL_PAD // NW
POOL_NCHUNK = POOL_RPW // POOL_CH
POOL_ACC_ROWS = G + 16


@functools.lru_cache(maxsize=None)
def _sc_kernels():
    mesh = plsc.VectorSubcoreMesh(core_axis_name="c", subcore_axis_name="s")
    cp = pltpu.CompilerParams(use_tc_tiling_on_sc=False)

    @functools.partial(
        pl.kernel,
        out_type=jax.ShapeDtypeStruct((NC, N, H), jnp.float32),
        mesh=mesh,
        compiler_params=cp,
        scratch_types=[
            pltpu.VMEM((NCHUNK, 1, CH), jnp.int32),
            pltpu.VMEM((NCHUNK, 1, CH), jnp.int32),
            pltpu.VMEM((NRING, CH, H), jnp.float32),
            pltpu.VMEM_SHARED((ACC_ROWS, H), jnp.float32),
            pltpu.SemaphoreType.DMA((2,)),
            pltpu.SemaphoreType.DMA((NRING,)),
            pltpu.SemaphoreType.DMA((NRING,)),
        ],
    )
    def agg_sc(z_hbm, src_hbm, dst_hbm, out_hbm, sidx, didx, rows, acc,
               isem, gsem, ssem):
        cid = lax.axis_index("c")
        sid = lax.axis_index("s")
        wid = sid * NC + cid
        r0 = sid * RS
        pltpu.async_copy(src_hbm.at[wid], sidx, isem.at[0])
        pltpu.async_copy(dst_hbm.at[wid], didx, isem.at[1])
        pltpu.sync_copy(z_hbm.at[pl.ds(r0, RS)], acc.at[pl.ds(r0, RS)])

        @pl.when(sid == NS - 1)
        def _():
            pltpu.sync_copy(z_hbm.at[pl.ds(NS * RS, RSREM)],
                            acc.at[pl.ds(NS * RS, RSREM)])

        plsc.subcore_barrier()
        pltpu.make_async_copy(src_hbm.at[wid], sidx, isem.at[0]).wait()
        pltpu.make_async_copy(dst_hbm.at[wid], didx, isem.at[1]).wait()

        for b in range(NBUF):
            pltpu.async_copy(z_hbm.at[sidx.at[b, 0]], rows.at[b], gsem.at[b])

        @pl.loop(0, NCHUNK, step=NRING)
        def _(j0):
            for b in range(NRING):
                j = j0 + b
                pltpu.make_async_copy(
                    z_hbm.at[pl.ds(0, CH)], rows.at[b], gsem.at[b]).wait()
                pltpu.async_copy(rows.at[b], acc.at[didx.at[j, 0]],
                                 ssem.at[b], add=True)
                b2 = (b + NBUF) % NRING
                if b2 < NBUF:
                    pltpu.make_async_copy(
                        z_hbm.at[pl.ds(0, CH)], rows.at[b2],
                        ssem.at[b2]).wait()
                else:
                    @pl.when(j0 > 0)
                    def _():
                        pltpu.make_async_copy(
                            z_hbm.at[pl.ds(0, CH)], rows.at[b2],
                            ssem.at[b2]).wait()

                @pl.when(j + NBUF < NCHUNK)
                def _():
                    pltpu.async_copy(z_hbm.at[sidx.at[j + NBUF, 0]],
                                     rows.at[b2], gsem.at[b2])

        for b in range(NBUF, NRING):
            pltpu.make_async_copy(
                z_hbm.at[pl.ds(0, CH)], rows.at[b], ssem.at[b]).wait()

        plsc.subcore_barrier()
        pltpu.sync_copy(acc.at[pl.ds(r0, RS)],
                        out_hbm.at[cid, pl.ds(r0, RS)])

        @pl.when(sid == NS - 1)
        def _():
            pltpu.sync_copy(acc.at[pl.ds(NS * RS, RSREM)],
                            out_hbm.at[cid, pl.ds(NS * RS, RSREM)])

    @functools.partial(
        pl.kernel,
        out_type=jax.ShapeDtypeStruct((NC, POOL_ACC_ROWS, PW), jnp.float32),
        mesh=mesh,
        compiler_params=cp,
        scratch_types=[
            pltpu.VMEM((1, POOL_CH), jnp.int32),
            pltpu.VMEM((POOL_CH, PW), jnp.float32),
            pltpu.VMEM((16, PW), jnp.float32),
            pltpu.VMEM_SHARED((POOL_ACC_ROWS, PW), jnp.float32),
            pltpu.SemaphoreType.DMA,
        ],
    )
    def pool_sc(xf_hbm, b_hbm, out_hbm, bidx, rows, zbuf, acc, sem):
        cid = lax.axis_index("c")
        sid = lax.axis_index("s")
        wid = sid * NC + cid
        a0 = sid * 8

        @pl.loop(0, 16)
        def _(i):
            zbuf[i] = jnp.zeros((PW,), jnp.float32)

        pltpu.sync_copy(zbuf.at[pl.ds(0, 8)], acc.at[pl.ds(a0, 8)])

        @pl.when(sid == 0)
        def _():
            pltpu.sync_copy(zbuf, acc.at[pl.ds(G, 16)])

        plsc.subcore_barrier()

        @pl.loop(0, POOL_NCHUNK)
        def _(j):
            base = wid * POOL_RPW + j * POOL_CH
            pltpu.sync_copy(xf_hbm.at[pl.ds(base, POOL_CH)], rows)
            pltpu.sync_copy(b_hbm.at[wid, j], bidx)
            pltpu.sync_copy(rows, acc.at[bidx.at[0]], add=True)

        plsc.subcore_barrier()
        pltpu.sync_copy(acc.at[pl.ds(a0, 8)], out_hbm.at[cid, pl.ds(a0, 8)])

        @pl.when(sid == 0)
        def _():
            pltpu.sync_copy(acc.at[pl.ds(G, 16)],
                            out_hbm.at[cid, pl.ds(G, 16)])

    return agg_sc, pool_sc


_PREC = lax.Precision.HIGHEST


def _dot(a, b):
    return jnp.dot(a, b, precision=_PREC, preferred_element_type=jnp.float32)


def _bn_relu(h, g, b):
    m = jnp.mean(h, axis=0, keepdims=True)
    c = h - m
    v = jnp.mean(c * c, axis=0, keepdims=True)
    return jnp.maximum(c / jnp.sqrt(v + 1e-5) * g + b, 0.0)


def _stage_a_body(x_ref, wa_ref, wb_ref, z1_ref, xp_ref):
    x = x_ref[...]
    z1_ref[...] = _dot(x, wa_ref[...])
    xp_ref[...] = _dot(x, wb_ref[...])


def _p1_body(z_ref, pp_ref, b1, g1, be1, w2, b2, t_ref):
    s = pp_ref[0] + pp_ref[1] - z_ref[...] + b1[...]
    a = _bn_relu(s, g1[...], be1[...])
    t_ref[...] = _dot(a, w2[...]) + b2[...]


def _p2_b_body(t_ref, g2, be2, wn, z2_ref):
    h = _bn_relu(t_ref[...], g2[...], be2[...])
    z2_ref[...] = _dot(h, wn[...])


def _p2_c_body(t_ref, g2, be2, xp_ref, wb, wnb, z3_ref, x2w_ref):
    h = _bn_relu(t_ref[...], g2[...], be2[...])
    z3_ref[...] = xp_ref[...] + _dot(h, wb[...])
    x2w_ref[...] = _dot(h, wnb[...])


def _p2_d_body(t_ref, g2, be2, x2w_ref, wa, z4_ref):
    h = _bn_relu(t_ref[...], g2[...], be2[...])
    z4_ref[...] = _dot(h, wa[...]) + x2w_ref[...]


def _p2_e_body(t_ref, g2, be2, wf, bf, xf_ref):
    h = _bn_relu(t_ref[...], g2[...], be2[...])
    xf_ref[0:N, :] = _dot(h, wf[...]) + bf[...]
    xf_ref[N:NPOOL_PAD, :] = jnp.zeros((NPOOL_PAD - N, PW), jnp.float32)


def _stage_f_body(pp_ref, out_ref):
    s = pp_ref[0] + pp_ref[1]
    cnt = jnp.maximum(s[:G, ONES_COL:ONES_COL + 1], 1.0)
    out_ref[...] = s[:G, :C] / cnt


def _f32(shape):
    return jax.ShapeDtypeStruct(shape, jnp.float32)


def kernel(x, edge_index, batch, params):
    src = edge_index[0].astype(jnp.int32)
    dst = edge_index[1].astype(jnp.int32)
    epw = E // NW
    pad_src = jnp.zeros((NW, EPW - epw), jnp.int32)
    pad_dst = jnp.broadcast_to(
        N + jnp.arange(EPW - epw, dtype=jnp.int32) % 16, (NW, EPW - epw))
    src3 = jnp.concatenate(
        [src.reshape(NW, epw), pad_src], axis=1).reshape(NW, NCHUNK, 1, CH)
    dst3 = jnp.concatenate(
        [dst.reshape(NW, epw), pad_dst], axis=1).reshape(NW, NCHUNK, 1, CH)
    batch3 = jnp.concatenate(
        [batch.astype(jnp.int32), jnp.full((NPOOL_PAD - N,), G, jnp.int32)]
    ).reshape(NW, POOL_NCHUNK, 1, POOL_CH)

    dc1, dc2, uc1, uc2 = (params['dc1'], params['dc2'], params['uc1'],
                          params['uc2'])

    def v1(p):
        return (p['b1'].reshape(1, H), p['g1'].reshape(1, H),
                p['be1'].reshape(1, H), p['W2'], p['b2'].reshape(1, H))

    def v2(p):
        return (p['g2'].reshape(1, H), p['be2'].reshape(1, H))

    agg_sc, pool_sc = _sc_kernels()

    wf = jnp.zeros((H, PW), jnp.float32).at[:, :C].set(params['final']['W'])
    bf = jnp.zeros((1, PW), jnp.float32).at[0, :C].set(
        params['final']['b']).at[0, ONES_COL].set(1.0)

    z1, xp = pl.pallas_call(
        _stage_a_body, out_shape=[_f32((N, H)), _f32((N, H))],
    )(x, dc1['W1'], uc1['W1'][:F])

    pp1 = agg_sc(z1, src3, dst3)
    t1 = pl.pallas_call(
        _p1_body, out_shape=_f32((N, H)))(z1, pp1, *v1(dc1))
    z2 = pl.pallas_call(
        _p2_b_body, out_shape=_f32((N, H)))(t1, *v2(dc1), dc2['W1'])

    pp2 = agg_sc(z2, src3, dst3)
    t2 = pl.pallas_call(
        _p1_body, out_shape=_f32((N, H)))(z2, pp2, *v1(dc2))
    z3, x2w = pl.pallas_call(
        _p2_c_body, out_shape=[_f32((N, H)), _f32((N, H))],
    )(t2, *v2(dc2), xp, uc1['W1'][F:], uc2['W1'][H:])

    pp3 = agg_sc(z3, src3, dst3)
    t3 = pl.pallas_call(
        _p1_body, out_shape=_f32((N, H)))(z3, pp3, *v1(uc1))
    z4 = pl.pallas_call(
        _p2_d_body, out_shape=_f32((N, H)))(t3, *v2(uc1), x2w, uc2['W1'][:H])

    pp4 = agg_sc(z4, src3, dst3)
    t4 = pl.pallas_call(
        _p1_body, out_shape=_f32((N, H)))(z4, pp4, *v1(uc2))
    xf = pl.pallas_call(
        _p2_e_body, out_shape=_f32((NPOOL_PAD, PW)))(t4, *v2(uc2), wf, bf)

    ppool = pool_sc(xf, batch3)
    return pl.pallas_call(_stage_f_body, out_shape=_f32((G, C)))(ppool)


# speedup vs baseline: 5.6877x; 1.0009x over previous
import functools

import jax
import jax.numpy as jnp
from jax import lax
from jax.experimental import pallas as pl
from jax.experimental.pallas import tpu as pltpu
from jax.experimental.pallas import tpu_sc as plsc

N = 10000
E = 160000
F = 256
H = 64
C = 10
G = 128

NC = 2
NS = 16
NW = NC * NS
CH = 128
EPW = 5120
NCHUNK = EPW // CH
EPAD = NW * EPW
NBUF = 4
NRING = 2 * NBUF
ACC_ROWS = N + 32 * 64
RS = 624
RSREM = N - NS * RS

PW = 16
ONES_COL = 10
NPOOL_PAD = 10240
POOL_CH = 64
POOL_RPW = NPOOL_PAD // NW
POOL_NCHUNK = POOL_RPW // POOL_CH
POOL_ACC_ROWS = G + 16


@functools.lru_cache(maxsize=None)
def _sc_kernels():
    mesh = plsc.VectorSubcoreMesh(core_axis_name="c", subcore_axis_name="s")
    cp = pltpu.CompilerParams(use_tc_tiling_on_sc=False)

    @functools.partial(
        pl.kernel,
        out_type=jax.ShapeDtypeStruct((NC, N, H), jnp.float32),
        mesh=mesh,
        compiler_params=cp,
        scratch_types=[
            pltpu.VMEM((NCHUNK, 1, CH), jnp.int32),
            pltpu.VMEM((NCHUNK, 1, CH), jnp.int32),
            pltpu.VMEM((NRING, CH, H), jnp.float32),
            pltpu.VMEM_SHARED((ACC_ROWS, H), jnp.float32),
            pltpu.SemaphoreType.DMA((2,)),
            pltpu.SemaphoreType.DMA((NRING,)),
            pltpu.SemaphoreType.DMA((NRING,)),
        ],
    )
    def agg_sc(z_hbm, src_hbm, dst_hbm, out_hbm, sidx, didx, rows, acc,
               isem, gsem, ssem):
        cid = lax.axis_index("c")
        sid = lax.axis_index("s")
        wid = sid * NC + cid
        r0 = sid * RS
        pltpu.async_copy(src_hbm.at[wid], sidx, isem.at[0])
        pltpu.async_copy(dst_hbm.at[wid], didx, isem.at[1])
        pltpu.sync_copy(z_hbm.at[pl.ds(r0, RS)], acc.at[pl.ds(r0, RS)])

        @pl.when(sid == NS - 1)
        def _():
            pltpu.sync_copy(z_hbm.at[pl.ds(NS * RS, RSREM)],
                            acc.at[pl.ds(NS * RS, RSREM)])

        plsc.subcore_barrier()
        pltpu.make_async_copy(src_hbm.at[wid], sidx, isem.at[0]).wait()
        pltpu.make_async_copy(dst_hbm.at[wid], didx, isem.at[1]).wait()

        for b in range(NBUF):
            pltpu.async_copy(z_hbm.at[sidx.at[b, 0]], rows.at[b], gsem.at[b])

        @pl.loop(0, NCHUNK, step=NRING)
        def _(j0):
            for b in range(NRING):
                j = j0 + b
                pltpu.make_async_copy(
                    z_hbm.at[pl.ds(0, CH)], rows.at[b], gsem.at[b]).wait()
                pltpu.async_copy(rows.at[b], acc.at[didx.at[j, 0]],
                                 ssem.at[b], add=True)
                b2 = (b + NBUF) % NRING
                if b2 < NBUF:
                    pltpu.make_async_copy(
                        z_hbm.at[pl.ds(0, CH)], rows.at[b2],
                        ssem.at[b2]).wait()
                else:
                    @pl.when(j0 > 0)
                    def _():
                        pltpu.make_async_copy(
                            z_hbm.at[pl.ds(0, CH)], rows.at[b2],
                            ssem.at[b2]).wait()

                @pl.when(j + NBUF < NCHUNK)
                def _():
                    pltpu.async_copy(z_hbm.at[sidx.at[j + NBUF, 0]],
                                     rows.at[b2], gsem.at[b2])

        for b in range(NBUF, NRING):
            pltpu.make_async_copy(
                z_hbm.at[pl.ds(0, CH)], rows.at[b], ssem.at[b]).wait()

        plsc.subcore_barrier()
        pltpu.sync_copy(acc.at[pl.ds(r0, RS)],
                        out_hbm.at[cid, pl.ds(r0, RS)])

        @pl.when(sid == NS - 1)
        def _():
            pltpu.sync_copy(acc.at[pl.ds(NS * RS, RSREM)],
                            out_hbm.at[cid, pl.ds(NS * RS, RSREM)])

    @functools.partial(
        pl.kernel,
        out_type=jax.ShapeDtypeStruct((NC, POOL_ACC_ROWS, PW), jnp.float32),
        mesh=mesh,
        compiler_params=cp,
        scratch_types=[
            pltpu.VMEM((1, POOL_CH), jnp.int32),
            pltpu.VMEM((POOL_CH, PW), jnp.float32),
            pltpu.VMEM((16, PW), jnp.float32),
            pltpu.VMEM_SHARED((POOL_ACC_ROWS, PW), jnp.float32),
            pltpu.SemaphoreType.DMA,
        ],
    )
    def pool_sc(xf_hbm, b_hbm, out_hbm, bidx, rows, zbuf, acc, sem):
        cid = lax.axis_index("c")
        sid = lax.axis_index("s")
        wid = sid * NC + cid
        a0 = sid * 8

        @pl.loop(0, 16)
        def _(i):
            zbuf[i] = jnp.zeros((PW,), jnp.float32)

        pltpu.sync_copy(zbuf.at[pl.ds(0, 8)], acc.at[pl.ds(a0, 8)])

        @pl.when(sid == 0)
        def _():
            pltpu.sync_copy(zbuf, acc.at[pl.ds(G, 16)])

        plsc.subcore_barrier()

        @pl.loop(0, POOL_NCHUNK)
        def _(j):
            base = wid * POOL_RPW + j * POOL_CH
            pltpu.sync_copy(xf_hbm.at[pl.ds(base, POOL_CH)], rows)
            pltpu.sync_copy(b_hbm.at[wid, j], bidx)
            pltpu.sync_copy(rows, acc.at[bidx.at[0]], add=True)

        plsc.subcore_barrier()
        pltpu.sync_copy(acc.at[pl.ds(a0, 8)], out_hbm.at[cid, pl.ds(a0, 8)])

        @pl.when(sid == 0)
        def _():
            pltpu.sync_copy(acc.at[pl.ds(G, 16)],
                            out_hbm.at[cid, pl.ds(G, 16)])

    return agg_sc, pool_sc


_PREC = lax.Precision.HIGHEST


def _dot(a, b):
    return jnp.dot(a, b, precision=_PREC, preferred_element_type=jnp.float32)


def _bn_relu(h, g, b):
    m = jnp.mean(h, axis=0, keepdims=True)
    c = h - m
    v = jnp.mean(c * c, axis=0, keepdims=True)
    return jnp.maximum(c / jnp.sqrt(v + 1e-5) * g + b, 0.0)


def _stage_a_body(x_ref, wa_ref, wb_ref, z1_ref, xp_ref):
    x = x_ref[...]
    z1_ref[...] = _dot(x, wa_ref[...])
    xp_ref[...] = _dot(x, wb_ref[...])


def _p1_body(z_ref, pp_ref, b1, g1, be1, w2, b2, t_ref):
    s = pp_ref[0] + pp_ref[1] - z_ref[...] + b1[...]
    a = _bn_relu(s, g1[...], be1[...])
    t_ref[...] = _dot(a, w2[...]) + b2[...]


def _p2_b_body(t_ref, g2, be2, wn, z2_ref):
    h = _bn_relu(t_ref[...], g2[...], be2[...])
    z2_ref[...] = _dot(h, wn[...])


def _p2_c_body(t_ref, g2, be2, xp_ref, wb, wnb, z3_ref, x2w_ref):
    h = _bn_relu(t_ref[...], g2[...], be2[...])
    z3_ref[...] = xp_ref[...] + _dot(h, wb[...])
    x2w_ref[...] = _dot(h, wnb[...])


def _p2_d_body(t_ref, g2, be2, x2w_ref, wa, z4_ref):
    h = _bn_relu(t_ref[...], g2[...], be2[...])
    z4_ref[...] = _dot(h, wa[...]) + x2w_ref[...]


def _p2_e_body(t_ref, g2, be2, wf, bf, xf_ref):
    h = _bn_relu(t_ref[...], g2[...], be2[...])
    xf_ref[0:N, :] = _dot(h, wf[...]) + bf[...]
    xf_ref[N:NPOOL_PAD, :] = jnp.zeros((NPOOL_PAD - N, PW), jnp.float32)


def _stage_f_body(pp_ref, out_ref):
    s = pp_ref[0] + pp_ref[1]
    cnt = jnp.maximum(s[:G, ONES_COL:ONES_COL + 1], 1.0)
    out_ref[...] = s[:G, :C] / cnt


def _f32(shape):
    return jax.ShapeDtypeStruct(shape, jnp.float32)


def kernel(x, edge_index, batch, params):
    src = edge_index[0].astype(jnp.int32)
    dst = edge_index[1].astype(jnp.int32)
    epw = E // NW
    pad_src = jnp.zeros((NW, EPW - epw), jnp.int32)
    pad_dst = (N + 64 * jnp.arange(NW, dtype=jnp.int32)[:, None]
               + jnp.arange(EPW - epw, dtype=jnp.int32)[None, :] % 64)
    src3 = jnp.concatenate(
        [src.reshape(NW, epw), pad_src], axis=1).reshape(NW, NCHUNK, 1, CH)
    dst3 = jnp.concatenate(
        [dst.reshape(NW, epw), pad_dst], axis=1).reshape(NW, NCHUNK, 1, CH)
    batch3 = jnp.concatenate(
        [batch.astype(jnp.int32), jnp.full((NPOOL_PAD - N,), G, jnp.int32)]
    ).reshape(NW, POOL_NCHUNK, 1, POOL_CH)

    dc1, dc2, uc1, uc2 = (params['dc1'], params['dc2'], params['uc1'],
                          params['uc2'])

    def v1(p):
        return (p['b1'].reshape(1, H), p['g1'].reshape(1, H),
                p['be1'].reshape(1, H), p['W2'], p['b2'].reshape(1, H))

    def v2(p):
        return (p['g2'].reshape(1, H), p['be2'].reshape(1, H))

    agg_sc, pool_sc = _sc_kernels()

    wf = jnp.zeros((H, PW), jnp.float32).at[:, :C].set(params['final']['W'])
    bf = jnp.zeros((1, PW), jnp.float32).at[0, :C].set(
        params['final']['b']).at[0, ONES_COL].set(1.0)

    z1, xp = pl.pallas_call(
        _stage_a_body, out_shape=[_f32((N, H)), _f32((N, H))],
    )(x, dc1['W1'], uc1['W1'][:F])

    pp1 = agg_sc(z1, src3, dst3)
    t1 = pl.pallas_call(
        _p1_body, out_shape=_f32((N, H)))(z1, pp1, *v1(dc1))
    z2 = pl.pallas_call(
        _p2_b_body, out_shape=_f32((N, H)))(t1, *v2(dc1), dc2['W1'])

    pp2 = agg_sc(z2, src3, dst3)
    t2 = pl.pallas_call(
        _p1_body, out_shape=_f32((N, H)))(z2, pp2, *v1(dc2))
    z3, x2w = pl.pallas_call(
        _p2_c_body, out_shape=[_f32((N, H)), _f32((N, H))],
    )(t2, *v2(dc2), xp, uc1['W1'][F:], uc2['W1'][H:])

    pp3 = agg_sc(z3, src3, dst3)
    t3 = pl.pallas_call(
        _p1_body, out_shape=_f32((N, H)))(z3, pp3, *v1(uc1))
    z4 = pl.pallas_call(
        _p2_d_body, out_shape=_f32((N, H)))(t3, *v2(uc1), x2w, uc2['W1'][:H])

    pp4 = agg_sc(z4, src3, dst3)
    t4 = pl.pallas_call(
        _p1_body, out_shape=_f32((N, H)))(z4, pp4, *v1(uc2))
    xf = pl.pallas_call(
        _p2_e_body, out_shape=_f32((NPOOL_PAD, PW)))(t4, *v2(uc2), wf, bf)

    ppool = pool_sc(xf, batch3)
    return pl.pallas_call(_stage_f_body, out_shape=_f32((G, C)))(ppool)


# speedup vs baseline: 9.6939x; 1.7044x over previous
import functools

import jax
import jax.numpy as jnp
from jax import lax
from jax.experimental import pallas as pl
from jax.experimental.pallas import tpu as pltpu
from jax.experimental.pallas import tpu_sc as plsc

N = 10000
E = 160000
F = 256
H = 64
C = 10
G = 128

NC = 2
NS = 16
NW = NC * NS
CH = 128
EPW = 5120
NCHUNK = EPW // CH
EPAD = NW * EPW
NBUF = 4
NRING = 2 * NBUF
ACC_ROWS = N + 32 * 64
RS = 624
RSREM = N - NS * RS

PW = 16
ONES_COL = 10
NPOOL_PAD = 10240
POOL_CH = 64
POOL_RPW = NPOOL_PAD // NW
POOL_NCHUNK = POOL_RPW // POOL_CH
POOL_ACC_ROWS = G + 16


@functools.lru_cache(maxsize=None)
def _sc_kernels():
    mesh = plsc.VectorSubcoreMesh(core_axis_name="c", subcore_axis_name="s")
    cp = pltpu.CompilerParams(use_tc_tiling_on_sc=False)

    @functools.partial(
        pl.kernel,
        out_type=jax.ShapeDtypeStruct((NC, N, H), jnp.float32),
        mesh=mesh,
        compiler_params=cp,
        scratch_types=[
            pltpu.VMEM((NCHUNK, 1, CH), jnp.int32),
            pltpu.VMEM((NCHUNK, 1, CH), jnp.int32),
            pltpu.VMEM((NRING, CH, H), jnp.float32),
            pltpu.VMEM_SHARED((ACC_ROWS, H), jnp.float32),
            pltpu.SemaphoreType.DMA((2,)),
            pltpu.SemaphoreType.DMA((NRING,)),
            pltpu.SemaphoreType.DMA((NRING,)),
        ],
    )
    def agg_sc(z_hbm, src_hbm, dst_hbm, out_hbm, sidx, didx, rows, acc,
               isem, gsem, ssem):
        cid = lax.axis_index("c")
        sid = lax.axis_index("s")
        wid = sid * NC + cid
        r0 = sid * RS
        pltpu.async_copy(src_hbm.at[wid], sidx, isem.at[0])
        pltpu.async_copy(dst_hbm.at[wid], didx, isem.at[1])
        pltpu.sync_copy(z_hbm.at[pl.ds(r0, RS)], acc.at[pl.ds(r0, RS)])

        @pl.when(sid == NS - 1)
        def _():
            pltpu.sync_copy(z_hbm.at[pl.ds(NS * RS, RSREM)],
                            acc.at[pl.ds(NS * RS, RSREM)])

        plsc.subcore_barrier()
        pltpu.make_async_copy(src_hbm.at[wid], sidx, isem.at[0]).wait()
        pltpu.make_async_copy(dst_hbm.at[wid], didx, isem.at[1]).wait()

        for b in range(NBUF):
            pltpu.async_copy(z_hbm.at[sidx.at[b, 0]], rows.at[b], gsem.at[b])

        @pl.loop(0, NCHUNK, step=NRING)
        def _(j0):
            for b in range(NRING):
                j = j0 + b
                pltpu.make_async_copy(
                    z_hbm.at[pl.ds(0, CH)], rows.at[b], gsem.at[b]).wait()
                pltpu.async_copy(rows.at[b], acc.at[didx.at[j, 0]],
                                 ssem.at[b], add=True)
                b2 = (b + NBUF) % NRING
                if b2 < NBUF:
                    pltpu.make_async_copy(
                        z_hbm.at[pl.ds(0, CH)], rows.at[b2],
                        ssem.at[b2]).wait()
                else:
                    @pl.when(j0 > 0)
                    def _():
                        pltpu.make_async_copy(
                            z_hbm.at[pl.ds(0, CH)], rows.at[b2],
                            ssem.at[b2]).wait()

                @pl.when(j + NBUF < NCHUNK)
                def _():
                    pltpu.async_copy(z_hbm.at[sidx.at[j + NBUF, 0]],
                                     rows.at[b2], gsem.at[b2])

        for b in range(NBUF, NRING):
            pltpu.make_async_copy(
                z_hbm.at[pl.ds(0, CH)], rows.at[b], ssem.at[b]).wait()

        plsc.subcore_barrier()
        pltpu.sync_copy(acc.at[pl.ds(r0, RS)],
                        out_hbm.at[cid, pl.ds(r0, RS)])

        @pl.when(sid == NS - 1)
        def _():
            pltpu.sync_copy(acc.at[pl.ds(NS * RS, RSREM)],
                            out_hbm.at[cid, pl.ds(NS * RS, RSREM)])

    @functools.partial(
        pl.kernel,
        out_type=jax.ShapeDtypeStruct((NC, POOL_ACC_ROWS, PW), jnp.float32),
        mesh=mesh,
        compiler_params=cp,
        scratch_types=[
            pltpu.VMEM((1, POOL_CH), jnp.int32),
            pltpu.VMEM((POOL_CH, PW), jnp.float32),
            pltpu.VMEM((16, PW), jnp.float32),
            pltpu.VMEM_SHARED((POOL_ACC_ROWS, PW), jnp.float32),
            pltpu.SemaphoreType.DMA,
        ],
    )
    def pool_sc(xf_hbm, b_hbm, out_hbm, bidx, rows, zbuf, acc, sem):
        cid = lax.axis_index("c")
        sid = lax.axis_index("s")
        wid = sid * NC + cid
        a0 = sid * 8

        @pl.loop(0, 16)
        def _(i):
            zbuf[i] = jnp.zeros((PW,), jnp.float32)

        pltpu.sync_copy(zbuf.at[pl.ds(0, 8)], acc.at[pl.ds(a0, 8)])

        @pl.when(sid == 0)
        def _():
            pltpu.sync_copy(zbuf, acc.at[pl.ds(G, 16)])

        plsc.subcore_barrier()

        @pl.loop(0, POOL_NCHUNK)
        def _(j):
            base = wid * POOL_RPW + j * POOL_CH
            pltpu.sync_copy(xf_hbm.at[pl.ds(base, POOL_CH)], rows)
            pltpu.sync_copy(b_hbm.at[wid, j], bidx)
            pltpu.sync_copy(rows, acc.at[bidx.at[0]], add=True)

        plsc.subcore_barrier()
        pltpu.sync_copy(acc.at[pl.ds(a0, 8)], out_hbm.at[cid, pl.ds(a0, 8)])

        @pl.when(sid == 0)
        def _():
            pltpu.sync_copy(acc.at[pl.ds(G, 16)],
                            out_hbm.at[cid, pl.ds(G, 16)])

    return agg_sc, pool_sc


_PREC = lax.Precision.HIGHEST


def _dot(a, b):
    return jnp.dot(a, b, precision=_PREC, preferred_element_type=jnp.float32)


def _bn_relu(h, g, b):
    m = jnp.mean(h, axis=0, keepdims=True)
    c = h - m
    v = jnp.mean(c * c, axis=0, keepdims=True)
    return jnp.maximum(c / jnp.sqrt(v + 1e-5) * g + b, 0.0)


def _stage_a_body(x_ref, wa_ref, wb_ref, z1_ref, xp_ref):
    x = x_ref[...]
    z1_ref[...] = _dot(x, wa_ref[...])
    xp_ref[...] = _dot(x, wb_ref[...])


def _p1_body(z_ref, pp_ref, b1, g1, be1, w2, b2, t_ref):
    s = pp_ref[0] + pp_ref[1] - z_ref[...] + b1[...]
    a = _bn_relu(s, g1[...], be1[...])
    t_ref[...] = _dot(a, w2[...]) + b2[...]


def _p2_b_body(t_ref, g2, be2, wn, z2_ref):
    h = _bn_relu(t_ref[...], g2[...], be2[...])
    z2_ref[...] = _dot(h, wn[...])


def _p2_c_body(t_ref, g2, be2, xp_ref, wb, wnb, z3_ref, x2w_ref):
    h = _bn_relu(t_ref[...], g2[...], be2[...])
    z3_ref[...] = xp_ref[...] + _dot(h, wb[...])
    x2w_ref[...] = _dot(h, wnb[...])


def _p2_d_body(t_ref, g2, be2, x2w_ref, wa, z4_ref):
    h = _bn_relu(t_ref[...], g2[...], be2[...])
    z4_ref[...] = _dot(h, wa[...]) + x2w_ref[...]


def _p2_e_body(t_ref, g2, be2, wf, bf, xf_ref):
    h = _bn_relu(t_ref[...], g2[...], be2[...])
    xf_ref[0:N, :] = _dot(h, wf[...]) + bf[...]
    xf_ref[N:NPOOL_PAD, :] = jnp.zeros((NPOOL_PAD - N, PW), jnp.float32)


def _stage_f_body(pp_ref, out_ref):
    s = pp_ref[0] + pp_ref[1]
    cnt = jnp.maximum(s[:G, ONES_COL:ONES_COL + 1], 1.0)
    out_ref[...] = s[:G, :C] / cnt


def _f32(shape):
    return jax.ShapeDtypeStruct(shape, jnp.float32)


def kernel(x, edge_index, batch, params):
    src = edge_index[0].astype(jnp.int32)
    dst = edge_index[1].astype(jnp.int32)
    epw = E // NW
    pad_src = jnp.broadcast_to(
        jnp.arange(EPW - epw, dtype=jnp.int32) * 64, (NW, EPW - epw))
    pad_dst = (N + 64 * jnp.arange(NW, dtype=jnp.int32)[:, None]
               + jnp.arange(EPW - epw, dtype=jnp.int32)[None, :] % 64)
    src3 = jnp.concatenate(
        [src.reshape(NW, epw), pad_src], axis=1).reshape(NW, NCHUNK, 1, CH)
    dst3 = jnp.concatenate(
        [dst.reshape(NW, epw), pad_dst], axis=1).reshape(NW, NCHUNK, 1, CH)
    batch3 = jnp.concatenate(
        [batch.astype(jnp.int32), jnp.full((NPOOL_PAD - N,), G, jnp.int32)]
    ).reshape(NW, POOL_NCHUNK, 1, POOL_CH)

    dc1, dc2, uc1, uc2 = (params['dc1'], params['dc2'], params['uc1'],
                          params['uc2'])

    def v1(p):
        return (p['b1'].reshape(1, H), p['g1'].reshape(1, H),
                p['be1'].reshape(1, H), p['W2'], p['b2'].reshape(1, H))

    def v2(p):
        return (p['g2'].reshape(1, H), p['be2'].reshape(1, H))

    agg_sc, pool_sc = _sc_kernels()

    wf = jnp.zeros((H, PW), jnp.float32).at[:, :C].set(params['final']['W'])
    bf = jnp.zeros((1, PW), jnp.float32).at[0, :C].set(
        params['final']['b']).at[0, ONES_COL].set(1.0)

    z1, xp = pl.pallas_call(
        _stage_a_body, out_shape=[_f32((N, H)), _f32((N, H))],
    )(x, dc1['W1'], uc1['W1'][:F])

    pp1 = agg_sc(z1, src3, dst3)
    t1 = pl.pallas_call(
        _p1_body, out_shape=_f32((N, H)))(z1, pp1, *v1(dc1))
    z2 = pl.pallas_call(
        _p2_b_body, out_shape=_f32((N, H)))(t1, *v2(dc1), dc2['W1'])

    pp2 = agg_sc(z2, src3, dst3)
    t2 = pl.pallas_call(
        _p1_body, out_shape=_f32((N, H)))(z2, pp2, *v1(dc2))
    z3, x2w = pl.pallas_call(
        _p2_c_body, out_shape=[_f32((N, H)), _f32((N, H))],
    )(t2, *v2(dc2), xp, uc1['W1'][F:], uc2['W1'][H:])

    pp3 = agg_sc(z3, src3, dst3)
    t3 = pl.pallas_call(
        _p1_body, out_shape=_f32((N, H)))(z3, pp3, *v1(uc1))
    z4 = pl.pallas_call(
        _p2_d_body, out_shape=_f32((N, H)))(t3, *v2(uc1), x2w, uc2['W1'][:H])

    pp4 = agg_sc(z4, src3, dst3)
    t4 = pl.pallas_call(
        _p1_body, out_shape=_f32((N, H)))(z4, pp4, *v1(uc2))
    xf = pl.pallas_call(
        _p2_e_body, out_shape=_f32((NPOOL_PAD, PW)))(t4, *v2(uc2), wf, bf)

    ppool = pool_sc(xf, batch3)
    return pl.pallas_call(_stage_f_body, out_shape=_f32((G, C)))(ppool)
